# baseline (speedup 1.0000x reference)
import jax
import jax.numpy as jnp
from jax.experimental import pallas as pl
from jax.experimental.pallas import tpu as pltpu

_BN = 512
_NFIX = 6
_TAU = 4e-3


def _acc_row_sum(v):
    err = jnp.zeros((v.shape[0], 1), jnp.float32)
    w = v.shape[1]
    while w > 1:
        h = w // 2
        a = v[:, :h]
        b = v[:, h:w]
        s = a + b
        ap = s - b
        bp = s - ap
        e = (a - ap) + (b - bp)
        err = err + jnp.sum(e, axis=1, keepdims=True)
        v = s
        w = h
    return v + err


def _tile_kernel(x_ref, c_ref, dist_ref, assign_ref, c2_ref, ch_ref, cl_ref):
    @pl.when(pl.program_id(0) == 0)
    def _():
        c = c_ref[...]
        c2_ref[...] = jnp.sum(c * c, axis=1, keepdims=True)
        ch = c.astype(jnp.bfloat16)
        ch_ref[...] = ch
        cl_ref[...] = (c - ch.astype(jnp.float32)).astype(jnp.bfloat16)

    x = x_ref[...]
    x2 = jnp.sum(x * x, axis=1)[None, :]
    xh = x.astype(jnp.bfloat16)
    xl = (x - xh.astype(jnp.float32)).astype(jnp.bfloat16)

    dims = (((1,), (1,)), ((), ()))
    f32 = jnp.float32
    dots = jax.lax.dot_general(ch_ref[...], xh, dims, preferred_element_type=f32)
    dots += jax.lax.dot_general(ch_ref[...], xl, dims, preferred_element_type=f32)
    dots += jax.lax.dot_general(cl_ref[...], xh, dims, preferred_element_type=f32)

    dist = (c2_ref[...] - 2.0 * dots) + x2
    dist_ref[...] = dist

    kk, bn = dist.shape
    rows = jax.lax.broadcasted_iota(jnp.int32, dist.shape, 0)
    ibig = jnp.int32(jnp.iinfo(jnp.int32).max)
    inf = jnp.float32(jnp.inf)

    d1 = jnp.min(dist, axis=0, keepdims=True)
    arg1 = jnp.min(jnp.where(dist == d1, rows, ibig), axis=0, keepdims=True)
    masked = jnp.where(rows == arg1, inf, dist)
    d2 = jnp.min(masked, axis=0, keepdims=True)
    arg2 = jnp.min(jnp.where(masked == d2, rows, ibig), axis=0, keepdims=True)

    margins = d2 - d1
    cols = jax.lax.broadcasted_iota(jnp.int32, (1, bn), 1)
    sel_rows = []
    for _ in range(_NFIX):
        mmin = jnp.min(margins, axis=1, keepdims=True)
        pcol = jnp.min(jnp.where(margins == mmin, cols, ibig),
                       axis=1, keepdims=True)
        hot = cols == pcol
        sel_rows.append(hot)
        margins = jnp.where(hot, inf, margins)
    self32 = jnp.concatenate([h.astype(f32) for h in sel_rows],
                             axis=0)

    a1f = jnp.sum(self32 * arg1.astype(f32), axis=1, keepdims=True)
    a2f = jnp.sum(self32 * arg2.astype(f32), axis=1, keepdims=True)
    a1 = a1f.astype(jnp.int32)
    a2 = a2f.astype(jnp.int32)

    hp = jax.lax.Precision.HIGHEST
    nd = (((1,), (0,)), ((), ()))
    xrow = jax.lax.dot_general(self32, x, nd,
                               preferred_element_type=f32,
                               precision=hp)
    crow_iota = jax.lax.broadcasted_iota(jnp.int32, (_NFIX, kk), 1)
    c = c_ref[...]
    c1row = jax.lax.dot_general((crow_iota == a1).astype(f32), c, nd,
                                preferred_element_type=f32, precision=hp)
    c2row = jax.lax.dot_general((crow_iota == a2).astype(f32), c, nd,
                                preferred_element_type=f32, precision=hp)

    dd1 = xrow - c1row
    dd2 = xrow - c2row
    e1 = _acc_row_sum(dd1 * dd1)
    e2 = _acc_row_sum(dd2 * dd2)
    pick2 = (e2 < e1) | ((e2 == e1) & (a2 < a1))
    fixed = jnp.where(pick2, a2, a1)

    assign = arg1
    for m in range(_NFIX):
        hot = sel_rows[m]
        assign = jnp.where(hot, fixed[m, 0], assign)
    assign_ref[...] = assign


def kernel(inputs, centroids):
    n, d = inputs.shape
    k, _ = centroids.shape
    bn = _BN
    dist, assign = pl.pallas_call(
        _tile_kernel,
        grid=(n // bn,),
        in_specs=[
            pl.BlockSpec((bn, d), lambda j: (j, 0)),
            pl.BlockSpec((k, d), lambda j: (0, 0)),
        ],
        out_specs=[
            pl.BlockSpec((k, bn), lambda j: (0, j)),
            pl.BlockSpec((1, bn), lambda j: (0, j)),
        ],
        out_shape=[
            jax.ShapeDtypeStruct((k, n), jnp.float32),
            jax.ShapeDtypeStruct((1, n), jnp.int32),
        ],
        scratch_shapes=[
            pltpu.VMEM((k, 1), jnp.float32),
            pltpu.VMEM((k, d), jnp.bfloat16),
            pltpu.VMEM((k, d), jnp.bfloat16),
        ],
        compiler_params=pltpu.CompilerParams(
            dimension_semantics=("arbitrary",)),
    )(inputs, centroids)
    return dist, assign[0]


# speedup vs baseline: 2.0997x; 2.0997x over previous
import jax
import jax.numpy as jnp
from jax.experimental import pallas as pl
from jax.experimental.pallas import tpu as pltpu

_BN = 512
_NFIX = 16
_TAU = 4e-3


def _acc_row_sum(v):
    err = jnp.zeros(v.shape, jnp.float32)
    w = v.shape[1]
    while w > 1:
        h = w // 2
        a = v[:, :h]
        b = v[:, h:w]
        s = a + b
        ap = s - b
        bp = s - ap
        e = (a - ap) + (b - bp)
        err = err[:, :h] + err[:, h:w] + e
        v = s
        w = h
    return v + err


def _tile_kernel(x_ref, c_ref, dist_ref, assign_ref,
                 c2_ref, ch_ref, cl_ref, cll_ref, tri_ref):
    f32 = jnp.float32
    bf16 = jnp.bfloat16
    i32 = jnp.int32

    @pl.when(pl.program_id(0) == 0)
    def _():
        c = c_ref[...]
        c2_ref[...] = jnp.sum(c * c, axis=1, keepdims=True)
        ch = c.astype(bf16)
        cl = (c - ch.astype(f32)).astype(bf16)
        ch_ref[...] = ch
        cl_ref[...] = cl
        cll_ref[...] = (c - ch.astype(f32) - cl.astype(f32)).astype(bf16)
        n_ = tri_ref.shape[0]
        tri_ref[...] = (jax.lax.broadcasted_iota(i32, (n_, n_), 0) <=
                        jax.lax.broadcasted_iota(i32, (n_, n_), 1)).astype(bf16)

    x = x_ref[...]
    x2 = jnp.sum(x * x, axis=1)[None, :]
    xh = x.astype(bf16)
    xl = (x - xh.astype(f32)).astype(bf16)

    dims = (((1,), (1,)), ((), ()))
    dots = jax.lax.dot_general(ch_ref[...], xh, dims, preferred_element_type=f32)
    dots += jax.lax.dot_general(ch_ref[...], xl, dims, preferred_element_type=f32)
    dots += jax.lax.dot_general(cl_ref[...], xh, dims, preferred_element_type=f32)

    dist = (c2_ref[...] - 2.0 * dots) + x2
    dist_ref[...] = dist

    kk, bn = dist.shape
    rows = jax.lax.broadcasted_iota(i32, dist.shape, 0)
    ibig = jnp.int32(jnp.iinfo(jnp.int32).max)
    inf = jnp.float32(jnp.inf)

    d1 = jnp.min(dist, axis=0, keepdims=True)
    arg1 = jnp.min(jnp.where(dist == d1, rows, ibig), axis=0, keepdims=True)
    assign_ref[...] = arg1

    near = (dist < d1 + _TAU).astype(i32)
    ncand = jnp.sum(near, axis=0, keepdims=True)
    flagged = ncand >= 2

    @pl.when(jnp.sum(ncand) > bn)
    def _refine():
        masked = jnp.where(rows == arg1, inf, dist)
        d2 = jnp.min(masked, axis=0, keepdims=True)
        arg2 = jnp.min(jnp.where(masked == d2, rows, ibig),
                       axis=0, keepdims=True)

        flagf = flagged.astype(bf16)
        rank = jax.lax.dot_general(flagf, tri_ref[...],
                                   (((1,), (0,)), ((), ())),
                                   preferred_element_type=f32)
        hots = [flagged & (rank == jnp.float32(m + 1)) for m in range(_NFIX)]
        self32 = jnp.concatenate([h.astype(f32) for h in hots], axis=0)
        selbf = jnp.concatenate([h.astype(bf16) for h in hots], axis=0)

        a1f = jnp.sum(self32 * arg1.astype(f32), axis=1, keepdims=True)
        a2f = jnp.sum(self32 * arg2.astype(f32), axis=1, keepdims=True)

        nd = (((1,), (0,)), ((), ()))
        xll = (x - xh.astype(f32) - xl.astype(f32)).astype(bf16)
        xrow = jax.lax.dot_general(selbf, xh, nd, preferred_element_type=f32)
        xrow += jax.lax.dot_general(selbf, xl, nd, preferred_element_type=f32)
        xrow += jax.lax.dot_general(selbf, xll, nd, preferred_element_type=f32)

        a12 = jnp.concatenate([a1f, a2f], axis=0)
        ohc = (jax.lax.broadcasted_iota(i32, (2 * _NFIX, kk), 1).astype(f32)
               == a12).astype(bf16)
        crow = jax.lax.dot_general(ohc, ch_ref[...], nd, preferred_element_type=f32)
        crow += jax.lax.dot_general(ohc, cl_ref[...], nd, preferred_element_type=f32)
        crow += jax.lax.dot_general(ohc, cll_ref[...], nd, preferred_element_type=f32)

        dd = jnp.concatenate([xrow, xrow], axis=0) - crow
        tot = _acc_row_sum(dd * dd)
        e1 = tot[:_NFIX]
        e2 = tot[_NFIX:]

        pick2 = (e2 < e1) | ((e2 == e1) & (a2f < a1f))
        fixedf = jnp.where(pick2, a2f, a1f)

        upd = jnp.sum(self32 * fixedf, axis=0, keepdims=True)
        anyhot = jnp.sum(self32, axis=0, keepdims=True) > 0.0
        assign_ref[...] = jnp.where(anyhot, upd.astype(i32), arg1)


def kernel(inputs, centroids):
    n, d = inputs.shape
    k, _ = centroids.shape
    bn = _BN
    dist, assign = pl.pallas_call(
        _tile_kernel,
        grid=(n // bn,),
        in_specs=[
            pl.BlockSpec((bn, d), lambda j: (j, 0)),
            pl.BlockSpec((k, d), lambda j: (0, 0)),
        ],
        out_specs=[
            pl.BlockSpec((k, bn), lambda j: (0, j)),
            pl.BlockSpec((1, bn), lambda j: (0, j)),
        ],
        out_shape=[
            jax.ShapeDtypeStruct((k, n), jnp.float32),
            jax.ShapeDtypeStruct((1, n), jnp.int32),
        ],
        scratch_shapes=[
            pltpu.VMEM((k, 1), jnp.float32),
            pltpu.VMEM((k, d), jnp.bfloat16),
            pltpu.VMEM((k, d), jnp.bfloat16),
            pltpu.VMEM((k, d), jnp.bfloat16),
            pltpu.VMEM((bn, bn), jnp.bfloat16),
        ],
        compiler_params=pltpu.CompilerParams(
            dimension_semantics=("arbitrary",)),
    )(inputs, centroids)
    return dist, assign[0]
